# baseline (speedup 1.0000x reference)
import functools

import jax
import jax.numpy as jnp
from jax import lax
from jax.experimental import pallas as pl
from jax.experimental.pallas import tpu as pltpu
from jax.experimental.pallas import tpu_sc as plsc

NUM_EMB = 1000000
DIM = 32
BATCH = 16384
HIST = 50

NC = 2
NS = 16
NW = NC * NS

NBLK = BATCH // 128
U_PER_W = NBLK // NW
NPAIR = HIST // 2

_mesh = plsc.VectorSubcoreMesh(core_axis_name="c", subcore_axis_name="s")


@functools.partial(
    pl.kernel,
    mesh=_mesh,
    compiler_params=pltpu.CompilerParams(use_tc_tiling_on_sc=False,
                                         needs_layout_passes=False),
    out_type=jax.ShapeDtypeStruct((HIST, DIM, BATCH), jnp.float32),
    scratch_types=[
        pltpu.VMEM((HIST, 128), jnp.int32),
        pltpu.VMEM((128,), jnp.int32),
        pltpu.VMEM((128,), jnp.int32),
        pltpu.VMEM((128, 128), jnp.float32),
        pltpu.VMEM((128, 128), jnp.float32),
        pltpu.VMEM((DIM, 128), jnp.float32),
        pltpu.VMEM((DIM, 128), jnp.float32),
        pltpu.SemaphoreType.DMA((2,)),
        pltpu.SemaphoreType.DMA((2,)),
    ],
)
def _gather_kernel(idx_hbm, staged_hbm, out_hbm, ibuf, jb0, jb1, r40, r41,
                   ot0, ot1, g_sem, s_sem):
    wid = lax.axis_index("s") * NC + lax.axis_index("c")
    iota16 = lax.iota(jnp.int32, 16)
    jb = (jb0, jb1)
    r4 = (r40, r41)
    ot = (ot0, ot1)

    def gather_fire(hp, par):
        for k in range(8):
            iv = plsc.load_gather(ibuf, [jnp.full((16,), hp, jnp.int32),
                                         k * 16 + iota16])
            jb[par][pl.ds(16 * k, 16)] = lax.shift_right_logical(iv, 2)
        pltpu.make_async_copy(staged_hbm.at[jb[par]], r4[par],
                              g_sem.at[par]).start()

    def select_store(hp, c0, par, cnt):
        pltpu.make_async_copy(staged_hbm.at[jb[par]], r4[par],
                              g_sem.at[par]).wait()

        @pl.when(cnt > 1)
        def _():
            pltpu.make_async_copy(ot[par], out_hbm.at[0, :, pl.ds(0, 128)],
                                  s_sem.at[par]).wait()

        for k in range(8):
            iv = plsc.load_gather(ibuf, [jnp.full((16,), hp, jnp.int32),
                                         k * 16 + iota16])
            colb = lax.shift_left(lax.bitwise_and(iv, 3), 5)
            rowi = k * 16 + iota16
            for f in range(DIM):
                g = plsc.load_gather(r4[par], [rowi, colb + f])
                ot[par][f, pl.ds(16 * k, 16)] = g
        pltpu.make_async_copy(ot[par], out_hbm.at[hp, :, pl.ds(c0, 128)],
                              s_sem.at[par]).start()

    def unit_body(n, cnt):
        c0 = (wid * U_PER_W + n) * 128
        pltpu.sync_copy(idx_hbm.at[:, pl.ds(c0, 128)], ibuf)

        def pair_body(i, cnt):
            h0 = 2 * i
            gather_fire(h0, 0)
            gather_fire(h0 + 1, 1)
            select_store(h0, c0, 0, cnt)
            select_store(h0 + 1, c0, 1, cnt + 1)
            return cnt + 2

        return lax.fori_loop(0, NPAIR, pair_body, cnt)

    lax.fori_loop(0, U_PER_W, unit_body, 0)

    for par in range(2):
        pltpu.make_async_copy(ot[par], out_hbm.at[0, :, pl.ds(0, 128)],
                              s_sem.at[par]).wait()


def kernel(inputs, table):
    idx_t = inputs.T.astype(jnp.int32)
    staged = table.reshape(NUM_EMB // 4, 4 * DIM)
    out_t = _gather_kernel(idx_t, staged)
    return jnp.transpose(out_t, (2, 0, 1))


# speedup vs baseline: 1.4604x; 1.4604x over previous
import functools

import jax
import jax.numpy as jnp
from jax import lax
from jax.experimental import pallas as pl
from jax.experimental.pallas import tpu as pltpu
from jax.experimental.pallas import tpu_sc as plsc

NUM_EMB = 1000000
DIM = 32
BATCH = 16384
HIST = 50

NC = 2
NS = 16
NW = NC * NS

NBLK = BATCH // 128
U_PER_W = NBLK // NW
NPAIR = HIST // 2

_mesh = plsc.VectorSubcoreMesh(core_axis_name="c", subcore_axis_name="s")


@functools.partial(
    pl.kernel,
    mesh=_mesh,
    compiler_params=pltpu.CompilerParams(use_tc_tiling_on_sc=False,
                                         needs_layout_passes=False),
    out_type=jax.ShapeDtypeStruct((HIST, BATCH, DIM), jnp.float32),
    scratch_types=[
        pltpu.VMEM((HIST, 128), jnp.int32),
        pltpu.VMEM((128,), jnp.int32),
        pltpu.VMEM((128,), jnp.int32),
        pltpu.VMEM((128, DIM), jnp.float32),
        pltpu.VMEM((128, DIM), jnp.float32),
        pltpu.SemaphoreType.DMA((2,)),
        pltpu.SemaphoreType.DMA((2,)),
    ],
)
def _gather_kernel(idx_hbm, table_hbm, out_hbm, ibuf, jb0, jb1, r0, r1,
                   g_sem, s_sem):
    wid = lax.axis_index("s") * NC + lax.axis_index("c")
    iota16 = lax.iota(jnp.int32, 16)
    jb = (jb0, jb1)
    rows = (r0, r1)

    def gather_fire(hp, par, cnt):
        @pl.when(cnt > 1)
        def _():
            pltpu.make_async_copy(rows[par], out_hbm.at[0, pl.ds(0, 128), :],
                                  s_sem.at[par]).wait()
        for k in range(8):
            iv = plsc.load_gather(ibuf, [jnp.full((16,), hp, jnp.int32),
                                         k * 16 + iota16])
            jb[par][pl.ds(16 * k, 16)] = iv
        pltpu.make_async_copy(table_hbm.at[jb[par]], rows[par],
                              g_sem.at[par]).start()

    def store_fire(hp, c0, par):
        pltpu.make_async_copy(table_hbm.at[jb[par]], rows[par],
                              g_sem.at[par]).wait()
        pltpu.make_async_copy(rows[par], out_hbm.at[hp, pl.ds(c0, 128), :],
                              s_sem.at[par]).start()

    def unit_body(n, cnt):
        c0 = (wid * U_PER_W + n) * 128
        pltpu.sync_copy(idx_hbm.at[:, pl.ds(c0, 128)], ibuf)

        def pair_body(i, cnt):
            h0 = 2 * i
            gather_fire(h0, 0, cnt)
            gather_fire(h0 + 1, 1, cnt + 1)
            store_fire(h0, c0, 0)
            store_fire(h0 + 1, c0, 1)
            return cnt + 2

        return lax.fori_loop(0, NPAIR, pair_body, cnt)

    lax.fori_loop(0, U_PER_W, unit_body, 0)

    for par in range(2):
        pltpu.make_async_copy(rows[par], out_hbm.at[0, pl.ds(0, 128), :],
                              s_sem.at[par]).wait()


def kernel(inputs, table):
    idx_t = inputs.T.astype(jnp.int32)
    out_t = _gather_kernel(idx_t, table)
    return jnp.transpose(out_t, (1, 0, 2))
